# baseline (speedup 1.0000x reference)
import functools

import jax
import jax.numpy as jnp
from jax import lax
from jax.experimental import pallas as pl
from jax.experimental.pallas import tpu as pltpu
from jax.experimental.pallas import tpu_sc as plsc

_BIG = 3.0e38


def _stage_a_body(xyz1t_ref, xyz2_ref, iw_ref, d_ref):
    x1 = xyz1t_ref[0]
    x2 = xyz2_ref[0]
    S = x2.shape[0]
    BN = x1.shape[1]
    b = pl.program_id(0)

    cross = jax.lax.dot_general(
        x2, -2.0 * x1, (((1,), (0,)), ((), ())),
        preferred_element_type=jnp.float32)
    x1sq = jnp.sum(x1 * x1, axis=0, keepdims=True)
    x2sq = jnp.sum(x2 * x2, axis=1, keepdims=True)
    d = cross + x1sq
    d = d + x2sq
    d_ref[...] = d
    d = d_ref[...]

    v1 = jnp.min(d, axis=0, keepdims=True)
    v2 = jnp.min(jnp.where(d <= v1, _BIG, d), axis=0, keepdims=True)
    v3 = jnp.min(jnp.where(d <= v2, _BIG, d), axis=0, keepdims=True)

    iota = jax.lax.broadcasted_iota(jnp.int32, (S, BN), 0).astype(jnp.float32)
    big_i = jnp.float32(S)
    i1 = jnp.min(jnp.where(d <= v1, iota, big_i), axis=0, keepdims=True)
    i2 = jnp.min(jnp.where((d <= v2) & (d > v1), iota, big_i),
                 axis=0, keepdims=True)
    i3 = jnp.min(jnp.where((d <= v3) & (d > v2), iota, big_i),
                 axis=0, keepdims=True)

    r1 = 1.0 / (v1 + 1e-8)
    r2 = 1.0 / (v2 + 1e-8)
    r3 = 1.0 / (v3 + 1e-8)
    rnorm = 1.0 / (r1 + r2 + r3)

    base = jnp.float32(S) * b
    iw_ref[0, 0:1, :] = i1 + base
    iw_ref[0, 1:2, :] = i2 + base
    iw_ref[0, 2:3, :] = i3 + base
    iw_ref[0, 3:4, :] = r1 * rnorm
    iw_ref[0, 4:5, :] = r2 * rnorm
    iw_ref[0, 5:6, :] = r3 * rnorm
    iw_ref[0, 6:7, :] = jnp.zeros_like(v1)
    iw_ref[0, 7:8, :] = jnp.zeros_like(v1)


def _stage_c_body(p1_ref, interp_ref, w1a_ref, w1b_ref, b1_ref,
                  w2_ref, b2_ref, out_ref):
    interpT = jnp.transpose(interp_ref[0], (1, 0))
    xA = p1_ref[0]
    h = jax.lax.dot_general(
        w1a_ref[...], xA, (((1,), (0,)), ((), ())),
        preferred_element_type=jnp.float32)
    h = h + jax.lax.dot_general(
        w1b_ref[...], interpT, (((1,), (0,)), ((), ())),
        preferred_element_type=jnp.float32)
    h = jnp.maximum(h + b1_ref[...], 0.0)
    o = jax.lax.dot_general(
        w2_ref[...], h, (((1,), (0,)), ((), ())),
        preferred_element_type=jnp.float32)
    o = jnp.maximum(o + b2_ref[...], 0.0)
    out_ref[0] = o


_NW = 32
_QC = 32


def _make_sc_gather(nq, D):
    q_per_w = nq // _NW
    n_chunks = q_per_w // _QC
    mesh = plsc.VectorSubcoreMesh(core_axis_name="c", subcore_axis_name="s")

    @functools.partial(
        pl.kernel, mesh=mesh,
        out_type=jax.ShapeDtypeStruct((nq, D), jnp.float32),
        scratch_types=[
            pltpu.VMEM((3 * _QC,), jnp.int32),
            pltpu.VMEM((3 * _QC, 16), jnp.float32),
            pltpu.VMEM((3 * _QC, D), jnp.float32),
            pltpu.VMEM((_QC, D), jnp.float32),
            pltpu.SemaphoreType.DMA,
        ],
    )
    def sc_gather(table_hbm, idx_hbm, wexp_hbm, out_hbm,
                  idx_v, w_v, rows_v, out_v, sem):
        wid = lax.axis_index("s") * 2 + lax.axis_index("c")
        qbase = wid * q_per_w

        def chunk(c, carry):
            q0 = qbase + c * _QC
            pltpu.sync_copy(idx_hbm.at[pl.ds(q0 * 3, 3 * _QC)], idx_v)
            pltpu.sync_copy(wexp_hbm.at[pl.ds(q0 * 3, 3 * _QC)], w_v)
            pltpu.async_copy(table_hbm.at[idx_v], rows_v, sem).wait()
            for q in range(_QC):
                w0 = w_v[3 * q, :]
                w1 = w_v[3 * q + 1, :]
                w2 = w_v[3 * q + 2, :]
                for j in range(D // 16):
                    sl = pl.ds(16 * j, 16)
                    acc = rows_v[3 * q, sl] * w0
                    acc = acc + rows_v[3 * q + 1, sl] * w1
                    acc = acc + rows_v[3 * q + 2, sl] * w2
                    out_v[q, sl] = acc
            pltpu.sync_copy(out_v, out_hbm.at[pl.ds(q0, _QC)])
            return carry

        lax.fori_loop(0, n_chunks, chunk, 0)

    return sc_gather


@jax.jit
def kernel(xyz1, xyz2, points1, points2, W1, b1, W2, b2):
    B, N, _ = xyz1.shape
    S = xyz2.shape[1]
    D = points1.shape[1]
    BN = 2048 if N % 2048 == 0 else N

    xyz1t = jnp.transpose(xyz1, (0, 2, 1))
    w1a = W1[:, :D]
    w1b = W1[:, D:]
    b1c = b1[:, None]
    b2c = b2[:, None]

    grid = (B, N // BN)
    iw = pl.pallas_call(
        _stage_a_body,
        grid=grid,
        in_specs=[
            pl.BlockSpec((1, 3, BN), lambda b, n: (b, 0, n)),
            pl.BlockSpec((1, S, 3), lambda b, n: (b, 0, 0)),
        ],
        out_specs=pl.BlockSpec((1, 8, BN), lambda b, n: (b, 0, n)),
        out_shape=jax.ShapeDtypeStruct((B, 8, N), jnp.float32),
        scratch_shapes=[pltpu.VMEM((S, BN), jnp.float32)],
    )(xyz1t, xyz2)

    idx_flat = jnp.transpose(iw[:, 0:3, :], (0, 2, 1)).reshape(B * N * 3)
    idx_flat = idx_flat.astype(jnp.int32)
    w_flat = jnp.transpose(iw[:, 3:6, :], (0, 2, 1)).reshape(B * N * 3)
    wexp = jnp.tile(w_flat[:, None], (1, 16))
    table = jnp.transpose(points2, (0, 2, 1)).reshape(B * S, D)

    interp = _make_sc_gather(B * N, D)(table, idx_flat, wexp)
    interp = interp.reshape(B, N, D)

    out = pl.pallas_call(
        _stage_c_body,
        grid=grid,
        in_specs=[
            pl.BlockSpec((1, D, BN), lambda b, n: (b, 0, n)),
            pl.BlockSpec((1, BN, D), lambda b, n: (b, n, 0)),
            pl.BlockSpec((D, D), lambda b, n: (0, 0)),
            pl.BlockSpec((D, D), lambda b, n: (0, 0)),
            pl.BlockSpec((D, 1), lambda b, n: (0, 0)),
            pl.BlockSpec((D, D), lambda b, n: (0, 0)),
            pl.BlockSpec((D, 1), lambda b, n: (0, 0)),
        ],
        out_specs=pl.BlockSpec((1, D, BN), lambda b, n: (b, 0, n)),
        out_shape=jax.ShapeDtypeStruct((B, D, N), jnp.float32),
    )(points1, interp, w1a, w1b, b1c, W2, b2c)
    return out


# speedup vs baseline: 5.1561x; 5.1561x over previous
import functools

import jax
import jax.numpy as jnp
from jax.experimental import pallas as pl
from jax.experimental.pallas import tpu as pltpu

_BIG = 3.0e38


def _fp_body(xyz1t_ref, xyz2_ref, p1_ref, p2_ref, w1a_ref, w1b_ref, b1_ref,
             w2_ref, b2_ref, out_ref, d_ref):
    x1 = xyz1t_ref[0]
    x2 = xyz2_ref[0]
    S = x2.shape[0]
    BN = x1.shape[1]

    cross = jax.lax.dot_general(
        x2, -2.0 * x1, (((1,), (0,)), ((), ())),
        preferred_element_type=jnp.float32)
    x1sq = jnp.sum(x1 * x1, axis=0, keepdims=True)
    x2sq = jnp.sum(x2 * x2, axis=1, keepdims=True)
    d = cross + x1sq
    d = d + x2sq
    d_ref[...] = d
    d = d_ref[...]

    t1 = d[0:8, :]
    t2 = jnp.full((8, BN), _BIG, jnp.float32)
    t3 = t2
    for i in range(1, S // 8):
        x = d[8 * i:8 * i + 8, :]
        a = jnp.minimum(t1, x)
        b = jnp.maximum(t1, x)
        c = jnp.minimum(t2, b)
        e = jnp.maximum(t2, b)
        t1 = a
        t3 = jnp.minimum(t3, e)
        t2 = c
    v1 = jnp.min(t1, axis=0, keepdims=True)
    v2 = jnp.minimum(
        jnp.min(jnp.where(t1 <= v1, _BIG, t1), axis=0, keepdims=True),
        jnp.min(t2, axis=0, keepdims=True))
    v3 = jnp.minimum(
        jnp.minimum(
            jnp.min(jnp.where(t1 <= v2, _BIG, t1), axis=0, keepdims=True),
            jnp.min(jnp.where(t2 <= v2, _BIG, t2), axis=0, keepdims=True)),
        jnp.min(t3, axis=0, keepdims=True))

    r1 = pl.reciprocal(v1 + 1e-8, approx=True)
    r2 = pl.reciprocal(v2 + 1e-8, approx=True)
    r3 = pl.reciprocal(v3 + 1e-8, approx=True)
    rnorm = 1.0 / (r1 + r2 + r3)

    onehotT = jnp.where(
        d <= v3,
        pl.reciprocal(d + 1e-8, approx=True) * rnorm,
        0.0)

    interpT = jax.lax.dot_general(
        p2_ref[0], onehotT,
        (((1,), (0,)), ((), ())),
        preferred_element_type=jnp.float32)

    xA = p1_ref[0]
    h = jax.lax.dot_general(
        w1a_ref[...], xA,
        (((1,), (0,)), ((), ())),
        preferred_element_type=jnp.float32)
    h = h + jax.lax.dot_general(
        w1b_ref[...], interpT,
        (((1,), (0,)), ((), ())),
        preferred_element_type=jnp.float32)
    h = jnp.maximum(h + b1_ref[...], 0.0)

    o = jax.lax.dot_general(
        w2_ref[...], h,
        (((1,), (0,)), ((), ())),
        preferred_element_type=jnp.float32)
    o = jnp.maximum(o + b2_ref[...], 0.0)
    out_ref[0] = o


@jax.jit
def kernel(xyz1, xyz2, points1, points2, W1, b1, W2, b2):
    B, N, _ = xyz1.shape
    S = xyz2.shape[1]
    D = points1.shape[1]
    BN = 2048 if N % 2048 == 0 else N

    xyz1t = jnp.transpose(xyz1, (0, 2, 1))
    w1a = W1[:, :D]
    w1b = W1[:, D:]
    b1c = b1[:, None]
    b2c = b2[:, None]

    grid = (B, N // BN)
    out = pl.pallas_call(
        _fp_body,
        grid=grid,
        in_specs=[
            pl.BlockSpec((1, 3, BN), lambda b, n: (b, 0, n)),
            pl.BlockSpec((1, S, 3), lambda b, n: (b, 0, 0)),
            pl.BlockSpec((1, D, BN), lambda b, n: (b, 0, n)),
            pl.BlockSpec((1, D, S), lambda b, n: (b, 0, 0)),
            pl.BlockSpec((D, D), lambda b, n: (0, 0)),
            pl.BlockSpec((D, D), lambda b, n: (0, 0)),
            pl.BlockSpec((D, 1), lambda b, n: (0, 0)),
            pl.BlockSpec((D, D), lambda b, n: (0, 0)),
            pl.BlockSpec((D, 1), lambda b, n: (0, 0)),
        ],
        out_specs=pl.BlockSpec((1, D, BN), lambda b, n: (b, 0, n)),
        out_shape=jax.ShapeDtypeStruct((B, D, N), jnp.float32),
        scratch_shapes=[pltpu.VMEM((S, BN), jnp.float32)],
    )(xyz1t, xyz2, points1, points2, w1a, w1b, b1c, W2, b2c)
    return out
